# Initial kernel scaffold; baseline (speedup 1.0000x reference)
import jax
import jax.numpy as jnp
from jax.experimental import pallas as pl


def kernel(x, topk_values, topk_indices, gate_logits, target):
    raise NotImplementedError("write your pallas kernel here")


# speedup vs baseline: 12.2012x; 12.2012x over previous
import functools

import jax
import jax.numpy as jnp
from jax.experimental import pallas as pl

_SIZE = 8192
_PAD = 0
_SMOOTH = 0.1
_CONF = 1.0 - _SMOOTH
_E = 8
_LOAD_COEF = 0.01
_Z_COEF = 0.001

_EPS = _SMOOTH / (_SIZE - 1)


def _loss_kernel(x_ref, tgt_ref, tv_ref, ti_ref, gl_ref, out_ref, *, rows, batch):
    g = pl.program_id(0)

    xb = x_ref[...]
    m = jnp.max(xb, axis=1, keepdims=True)
    s = jnp.sum(jnp.exp(xb - m), axis=1)
    lse = m[:, 0] + jnp.log(s)
    sumx = jnp.sum(xb, axis=1)

    t = tgt_ref[0, 0, :]
    cols = jax.lax.broadcasted_iota(jnp.int32, (rows, _SIZE), 1)
    xt = jnp.sum(jnp.where(cols == t[:, None], xb, 0.0), axis=1)

    c_const = _CONF * jnp.log(_CONF) + (_SIZE - 1) * _EPS * jnp.log(_EPS)
    valid = (t != _PAD).astype(jnp.float32)
    contrib = jnp.sum(valid * (c_const + lse - _EPS * sumx - (_CONF - _EPS) * xt))
    contrib = contrib / batch

    @pl.when(g == 0)
    def _init():
        tv = tv_ref[0, :]
        ti = ti_ref[0, :]
        load_dot = 0.0
        for e in range(_E):
            sel = ti == e
            loads_e = jnp.sum(sel.astype(jnp.float32))
            sums_e = jnp.sum(jnp.where(sel, tv, 0.0))
            load_dot = load_dot + loads_e * sums_e
        num_elements = tv.shape[0] // 2
        load_loss = (_E / num_elements) * load_dot

        gl = gl_ref[...]
        m8 = jnp.max(gl, axis=1, keepdims=True)
        z = m8[:, 0] + jnp.log(jnp.sum(jnp.exp(gl - m8), axis=1))
        z_loss = jnp.mean(z * z)

        total = _LOAD_COEF * load_loss + _Z_COEF * z_loss + contrib
        out_ref[...] = total.reshape(1, 1)

    @pl.when(g != 0)
    def _acc():
        out_ref[...] += contrib.reshape(1, 1)


def kernel(x, topk_values, topk_indices, gate_logits, target):
    batch = x.shape[0]
    x2 = x.reshape(-1, _SIZE)
    n = x2.shape[0]
    rows = 256
    grid = n // rows

    tgt = target.reshape(grid, 1, rows)
    tv = topk_values.reshape(1, -1)
    ti = topk_indices.reshape(1, -1).astype(jnp.int32)
    gl = gate_logits.reshape(-1, _E)

    out = pl.pallas_call(
        functools.partial(_loss_kernel, rows=rows, batch=batch),
        grid=(grid,),
        in_specs=[
            pl.BlockSpec((rows, _SIZE), lambda g: (g, 0)),
            pl.BlockSpec((1, 1, rows), lambda g: (g, 0, 0)),
            pl.BlockSpec(tv.shape, lambda g: (0, 0)),
            pl.BlockSpec(ti.shape, lambda g: (0, 0)),
            pl.BlockSpec(gl.shape, lambda g: (0, 0)),
        ],
        out_specs=pl.BlockSpec((1, 1), lambda g: (0, 0)),
        out_shape=jax.ShapeDtypeStruct((1, 1), jnp.float32),
    )(x2, tgt, tv, ti, gl)
    return out[0, 0]


# speedup vs baseline: 12.7296x; 1.0433x over previous
import functools

import jax
import jax.numpy as jnp
from jax.experimental import pallas as pl

_SIZE = 8192
_PAD = 0
_SMOOTH = 0.1
_CONF = 1.0 - _SMOOTH
_E = 8
_LOAD_COEF = 0.01
_Z_COEF = 0.001

_EPS = _SMOOTH / (_SIZE - 1)


def _loss_kernel(x_ref, tgt_ref, tv_ref, ti_ref, gl_ref, out_ref, *, rows, batch):
    g = pl.program_id(0)

    xb = x_ref[...]
    m = jnp.max(xb, axis=1, keepdims=True)
    s = jnp.sum(jnp.exp(xb - m), axis=1)
    lse = m[:, 0] + jnp.log(s)
    sumx = jnp.sum(xb, axis=1)

    t = tgt_ref[0, 0, :]
    cols = jax.lax.broadcasted_iota(jnp.int32, (rows, _SIZE), 1)
    xt = jnp.sum(jnp.where(cols == t[:, None], xb, 0.0), axis=1)

    c_const = _CONF * jnp.log(_CONF) + (_SIZE - 1) * _EPS * jnp.log(_EPS)
    valid = (t != _PAD).astype(jnp.float32)
    contrib = jnp.sum(valid * (c_const + lse - _EPS * sumx - (_CONF - _EPS) * xt))
    contrib = contrib / batch

    @pl.when(g == 0)
    def _init():
        tv = tv_ref[0, :]
        ti = ti_ref[0, :]
        load_dot = 0.0
        for e in range(_E):
            sel = ti == e
            loads_e = jnp.sum(sel.astype(jnp.float32))
            sums_e = jnp.sum(jnp.where(sel, tv, 0.0))
            load_dot = load_dot + loads_e * sums_e
        num_elements = tv.shape[0] // 2
        load_loss = (_E / num_elements) * load_dot

        gl = gl_ref[...]
        m8 = jnp.max(gl, axis=1, keepdims=True)
        z = m8[:, 0] + jnp.log(jnp.sum(jnp.exp(gl - m8), axis=1))
        z_loss = jnp.mean(z * z)

        total = _LOAD_COEF * load_loss + _Z_COEF * z_loss + contrib
        out_ref[...] = total.reshape(1, 1)

    @pl.when(g != 0)
    def _acc():
        out_ref[...] += contrib.reshape(1, 1)


def kernel(x, topk_values, topk_indices, gate_logits, target):
    batch = x.shape[0]
    x2 = x.reshape(-1, _SIZE)
    n = x2.shape[0]
    rows = 512
    grid = n // rows

    tgt = target.reshape(grid, 1, rows)
    tv = topk_values.reshape(1, -1)
    ti = topk_indices.reshape(1, -1).astype(jnp.int32)
    gl = gate_logits.reshape(-1, _E)

    out = pl.pallas_call(
        functools.partial(_loss_kernel, rows=rows, batch=batch),
        grid=(grid,),
        in_specs=[
            pl.BlockSpec((rows, _SIZE), lambda g: (g, 0)),
            pl.BlockSpec((1, 1, rows), lambda g: (g, 0, 0)),
            pl.BlockSpec(tv.shape, lambda g: (0, 0)),
            pl.BlockSpec(ti.shape, lambda g: (0, 0)),
            pl.BlockSpec(gl.shape, lambda g: (0, 0)),
        ],
        out_specs=pl.BlockSpec((1, 1), lambda g: (0, 0)),
        out_shape=jax.ShapeDtypeStruct((1, 1), jnp.float32),
    )(x2, tgt, tv, ti, gl)
    return out[0, 0]
